# baseline (speedup 1.0000x reference)
import functools

import jax
import jax.numpy as jnp
from jax import lax
from jax.experimental import pallas as pl
from jax.experimental.pallas import tpu as pltpu
from jax.experimental.pallas import tpu_sc as plsc

_LANES = 16


def _build_call(B, NP1, E, N, P, n_rows_pos, n_rows_time):
  NW = 32
  tokens = NP1 - 1
  tok_pw = tokens // NW
  n_pw = N // NW
  chunk = tok_pw * E
  row_pad = ((E + _LANES - 1) // _LANES) * _LANES
  nvec = chunk // _LANES
  nvec_row = row_pad // _LANES
  b_pw = B // NW

  assert tokens % NW == 0 and N % NW == 0 and chunk % _LANES == 0
  assert B % NW == 0

  mesh = plsc.VectorSubcoreMesh(core_axis_name="c", subcore_axis_name="s")

  @functools.partial(
      pl.kernel,
      mesh=mesh,
      out_type=jax.ShapeDtypeStruct((B * NP1 * E,), jnp.float32),
      scratch_types=[
          pltpu.VMEM((chunk,), jnp.float32),
          pltpu.VMEM((chunk,), jnp.float32),
          pltpu.VMEM((chunk,), jnp.float32),
          pltpu.VMEM((chunk,), jnp.float32),
          pltpu.VMEM((row_pad,), jnp.float32),
      ],
  )
  def sc_kernel(x_hbm, pos_hbm, time_hbm, out_hbm, comb, t4, buf0, buf1, prow):
    wid = lax.axis_index("s") * 2 + lax.axis_index("c")

    for j in range(n_pw):
      src = pl.multiple_of((1 + wid * n_pw + j) * E, 8)
      for p in range(P):
        dst = ((j * P + p) * E)
        pltpu.sync_copy(pos_hbm.at[pl.ds(src, E)], comb.at[pl.ds(dst, E)])
    for j in range(tok_pw // P):
      pltpu.sync_copy(time_hbm.at[pl.ds(0, P * E)],
                      t4.at[pl.ds(j * P * E, P * E)])

    def add_time(i, _):
      sl = pl.ds(i * _LANES, _LANES)
      comb[sl] = comb[sl] + t4[sl]
      return 0
    lax.fori_loop(0, nvec, add_time, 0)

    pltpu.sync_copy(pos_hbm.at[pl.ds(0, E)], prow.at[pl.ds(0, E)])
    for bi in range(b_pw):
      b = wid * b_pw + bi
      off = pl.multiple_of(b * NP1 * E, 8)
      pltpu.sync_copy(x_hbm.at[pl.ds(off, E)], buf0.at[pl.ds(0, E)])
      for i in range(nvec_row - 1):
        sl = pl.ds(i * _LANES, _LANES)
        buf0[sl] = buf0[sl] + prow[sl]
      sl = pl.ds((nvec_row - 1) * _LANES, _LANES)
      buf0[sl] = buf0[sl] + prow[sl]
      pltpu.sync_copy(buf0.at[pl.ds(0, E)], out_hbm.at[pl.ds(off, E)])

    t0_off = (1 + wid * tok_pw) * E

    def do_batch(b, buf):
      off = pl.multiple_of(b * NP1 * E + t0_off, 8)
      pltpu.sync_copy(x_hbm.at[pl.ds(off, chunk)], buf)

      def add_comb(i, _):
        sl = pl.ds(i * _LANES, _LANES)
        buf[sl] = buf[sl] + comb[sl]
        return 0
      lax.fori_loop(0, nvec, add_comb, 0)
      pltpu.sync_copy(buf, out_hbm.at[pl.ds(off, chunk)])

    def body(bb, _):
      do_batch(2 * bb, buf0)
      do_batch(2 * bb + 1, buf1)
      return 0
    lax.fori_loop(0, B // 2, body, 0)

  return sc_kernel


def kernel(x, N, P, pos_embed, time_embed):
  del N, P
  B, NP1, E = x.shape
  Ns = pos_embed.shape[0] - 1
  Ps = time_embed.shape[0]
  call = _build_call(B, NP1, E, Ns, Ps,
                     pos_embed.shape[0], time_embed.shape[0])
  out = call(x.reshape(-1), pos_embed.reshape(-1), time_embed.reshape(-1))
  return out.reshape(B, NP1, E)
